# baseline (speedup 1.0000x reference)
import functools

import jax
import jax.numpy as jnp
from jax import lax
from jax.experimental import pallas as pl
from jax.experimental.pallas import tpu as pltpu
import jax.experimental.pallas.tpu_sc as plsc

N = 10000
D = 128
E = 320000
DG = 64
NUM_CLS = 16

NC = 2
NS = 16
NW = NC * NS
L = 16
NPW = (N + NW - 1) // NW
NPAD = NW * NPW
C = 3200
NCHUNK = E // C
G = 128
DUMMY = NPW * DG
NEG_INF = float("-inf")


def _segmax_body(q_hbm, ei_hbm, out_hbm,
                 eb0, eb1, eb2, eb3,
                 mdl0, msrc0, mdl1, msrc1, mdl2, msrc2, mdl3, msrc3,
                 rows0, rows1, rows2, rows3, acc,
                 esem0, esem1, esem2, esem3,
                 gsem0, gsem1, gsem2, gsem3):
    wid = lax.axis_index("s") * NC + lax.axis_index("c")
    lo = wid * NPW
    hi = lo + NPW

    ebufs = (eb0, eb1, eb2, eb3)
    mdls = (mdl0, mdl1, mdl2, mdl3)
    msrcs = (msrc0, msrc1, msrc2, msrc3)
    rowss = (rows0, rows1, rows2, rows3)
    esems = (esem0, esem1, esem2, esem3)
    gsems = (gsem0, gsem1, gsem2, gsem3)

    def initacc_body(i, _):
        acc[pl.ds(i * L, L)] = jnp.full((L,), NEG_INF, jnp.float32)
        return 0
    lax.fori_loop(0, (NPW + 1) * DG // L, initacc_body, 0)

    def initsrc_body(i, _):
        z = jnp.zeros((L,), jnp.int32)
        for ms in msrcs:
            ms[pl.ds(i * L, L)] = z
        return 0
    lax.fori_loop(0, (C + L) // L, initsrc_body, 0)

    def fire_edges(c, par):
        pltpu.make_async_copy(ei_hbm.at[:, pl.ds(c * C, C)],
                              ebufs[par], esems[par]).start()

    def drain_edges(c, par):
        pltpu.make_async_copy(ei_hbm.at[:, pl.ds(c * C, C)],
                              ebufs[par], esems[par]).wait()

    def fire_rows(par, boff):
        pltpu.make_async_copy(
            q_hbm.at[msrcs[par].at[pl.ds(boff, G)]],
            rowss[par], gsems[par]).start()

    def drain_rows(par):
        pltpu.make_async_copy(
            q_hbm.at[msrcs[par].at[pl.ds(0, G)]],
            rowss[par], gsems[par]).wait()

    def scan_chunk(par):
        eb, mdl, msrc = ebufs[par], mdls[par], msrcs[par]

        def scan_body(it, cnt):
            for u in range(4):
                base = it * (4 * L) + u * L
                d = eb[1, pl.ds(base, L)]
                s = eb[0, pl.ds(base, L)]
                m = (d >= lo) & (d < hi)
                pc = plsc.all_reduce_population_count(m)
                pcs = pc[0] if pc.ndim else pc
                pos = plsc.cumsum(m.astype(jnp.int32))
                idx = cnt + pos - 1
                plsc.store_scatter(mdl, [idx], (d - lo) * DG, mask=m)
                plsc.store_scatter(msrc, [idx], s, mask=m)
                cnt = cnt + pcs
            return cnt

        cnt = lax.fori_loop(0, C // (4 * L), scan_body, jnp.int32(0))
        mdl[pl.ds(cnt, L)] = jnp.full((L,), DUMMY, jnp.int32)

        @pl.when(cnt > 0)
        def _():
            fire_rows(par, 0)
        return cnt

    def process_batch(par, boff, nedge):
        mdl, rows = mdls[par], rowss[par]
        nblk = (nedge + (L - 1)) // L

        def blk_body(blk, _):
            dv = mdl[pl.ds(boff + blk * L, L)]
            rbase = blk * L
            for i in range(L):
                off = dv[i]
                for j in range(DG // L):
                    sl = pl.ds(off + j * L, L)
                    acc[sl] = jnp.maximum(acc[sl], rows[rbase + i, pl.ds(j * L, L)])
            return 0
        lax.fori_loop(0, nblk, blk_body, 0)

    def process_chunk(par, cnt):
        @pl.when(cnt > 0)
        def _():
            drain_rows(par)
            process_batch(par, 0, jnp.minimum(cnt, G))
            nb = (cnt + (G - 1)) // G

            def more_body(b, _):
                fire_rows(par, b * G)
                drain_rows(par)
                process_batch(par, b * G, jnp.minimum(cnt - b * G, G))
                return 0
            lax.fori_loop(1, nb, more_body, 0)

    for j in range(4):
        fire_edges(j, j)
    cnts = []
    for j in range(3):
        drain_edges(j, j)
        cnts.append(scan_chunk(j))

    def quad_iter(i, h, carry):
        c0, c1, c2 = carry

        @pl.when(i + 4 < NCHUNK)
        def _():
            fire_edges(i + 4, h)

        def do_scan():
            drain_edges(i + 3, (h + 3) % 4)
            return scan_chunk((h + 3) % 4)
        cnt_new = lax.cond(i + 3 < NCHUNK, do_scan, lambda: jnp.int32(0))

        process_chunk(h, c0)
        return (c1, c2, cnt_new)

    def quad_body(k, carry):
        for h in range(4):
            carry = quad_iter(4 * k + h, h, carry)
        return carry

    lax.fori_loop(0, NCHUNK // 4, quad_body, tuple(cnts))

    pltpu.sync_copy(acc.at[pl.ds(0, NPW * DG)],
                    out_hbm.at[pl.ds(wid * NPW * DG, NPW * DG)])


_segmax = pl.kernel(
    _segmax_body,
    out_type=jax.ShapeDtypeStruct((NPAD * DG,), jnp.float32),
    mesh=plsc.VectorSubcoreMesh(core_axis_name="c", subcore_axis_name="s"),
    scratch_types=(
        [pltpu.VMEM((2, C), jnp.int32)] * 4
        + [pltpu.VMEM((C + L,), jnp.int32)] * 8
        + [pltpu.VMEM((G, DG), jnp.float32)] * 4
        + [pltpu.VMEM(((NPW + 1) * DG,), jnp.float32)]
        + [pltpu.SemaphoreType.DMA] * 8
    ),
    compiler_params=pltpu.CompilerParams(
        use_tc_tiling_on_sc=False, needs_layout_passes=False),
)


BR = 1000


def _tca_body(x_ref, we_ref, be_ref, w1_ref, b1_ref, p_ref, q_ref):
    h = jnp.maximum(x_ref[...] @ we_ref[...] + be_ref[...], 0.0)
    w1 = w1_ref[...]
    wt, wb = w1[:D], w1[D:]
    q_ref[...] = h @ wb
    p_ref[...] = h @ (wt - wb) + b1_ref[...]


def _epilogue(p, s):
    v = p + s
    v = jnp.where(v >= 0.0, v, 0.01 * v)
    return jnp.where(jnp.isfinite(v), v, 0.0)


def _tcb_body(p_ref, s_ref, w2_ref, b2_ref, p2_ref, q2_ref):
    h1 = _epilogue(p_ref[...], s_ref[...])
    w2 = w2_ref[...]
    wt, wb = w2[:DG], w2[DG:]
    q2_ref[...] = h1 @ wb
    p2_ref[...] = h1 @ (wt - wb) + b2_ref[...]


def _tcc_body(p_ref, s_ref, wc_ref, bc_ref, out_ref):
    h2 = _epilogue(p_ref[...], s_ref[...])
    out_ref[...] = h2 @ wc_ref[...] + bc_ref[...]


def _full(shape):
    return pl.BlockSpec(shape, lambda i: (0, 0))


def _rows(w):
    return pl.BlockSpec((BR, w), lambda i: (i, 0))


_tca = pl.pallas_call(
    _tca_body,
    grid=(N // BR,),
    in_specs=[_rows(D), _full((D, D)), _full((1, D)),
              _full((2 * D, DG)), _full((1, DG))],
    out_specs=[_rows(DG), _rows(DG)],
    out_shape=[jax.ShapeDtypeStruct((N, DG), jnp.float32)] * 2,
)

_tcb = pl.pallas_call(
    _tcb_body,
    grid=(N // BR,),
    in_specs=[_rows(DG), _rows(DG), _full((2 * DG, DG)), _full((1, DG))],
    out_specs=[_rows(DG), _rows(DG)],
    out_shape=[jax.ShapeDtypeStruct((N, DG), jnp.float32)] * 2,
)

_tcc = pl.pallas_call(
    _tcc_body,
    grid=(N // BR,),
    in_specs=[_rows(DG), _rows(DG), _full((DG, NUM_CLS)),
              _full((1, NUM_CLS))],
    out_specs=_rows(NUM_CLS),
    out_shape=jax.ShapeDtypeStruct((N, NUM_CLS), jnp.float32),
)


@jax.jit
def kernel(x, edge_index, W_embed, b_embed, W1, b1, W2, b2, Wc, bc):
    p1, q1 = _tca(x, W_embed, b_embed.reshape(1, D), W1, b1.reshape(1, DG))
    s1 = _segmax(q1, edge_index).reshape(NPAD, DG)[:N]
    p2, q2 = _tcb(p1, s1, W2, b2.reshape(1, DG))
    s2 = _segmax(q2, edge_index).reshape(NPAD, DG)[:N]
    return _tcc(p2, s2, Wc, bc.reshape(1, NUM_CLS))


# speedup vs baseline: 1.9896x; 1.9896x over previous
import functools

import jax
import jax.numpy as jnp
from jax import lax
from jax.experimental import pallas as pl
from jax.experimental.pallas import tpu as pltpu
import jax.experimental.pallas.tpu_sc as plsc

N = 10000
D = 128
E = 320000
DG = 64
NUM_CLS = 16

NC = 2
NS = 16
NW = NC * NS
L = 16
NPW = (N + NW - 1) // NW
NPAD = NW * NPW
C = 3200
NCHUNK = E // C
G = 128
DUMMY = NPW * DG
NEG_INF = float("-inf")


def _segmax_body(q_hbm, ei_hbm, out_hbm,
                 eb0, eb1, mdl0, msrc0, mdl1, msrc1, rows0, rows1, acc,
                 spq, esem0, esem1, gsem0, gsem1, ssem):
    wid = lax.axis_index("s") * NC + lax.axis_index("c")
    sid = lax.axis_index("s")
    lo = wid * NPW
    hi = lo + NPW

    ebufs = (eb0, eb1)
    mdls = (mdl0, mdl1)
    msrcs = (msrc0, msrc1)
    rowss = (rows0, rows1)
    esems = (esem0, esem1)
    gsems = (gsem0, gsem1)

    QS = N // NS
    pltpu.async_copy(q_hbm.at[pl.ds(sid * QS, QS)],
                     spq.at[pl.ds(sid * QS, QS)], ssem).wait()

    def initacc_body(i, _):
        acc[pl.ds(i * L, L)] = jnp.full((L,), NEG_INF, jnp.float32)
        return 0
    lax.fori_loop(0, (NPW + 1) * DG // L, initacc_body, 0)

    def initsrc_body(i, _):
        z = jnp.zeros((L,), jnp.int32)
        msrc0[pl.ds(i * L, L)] = z
        msrc1[pl.ds(i * L, L)] = z
        return 0
    lax.fori_loop(0, (C + L) // L, initsrc_body, 0)
    plsc.subcore_barrier()

    def fire_edges(c, par):
        pltpu.make_async_copy(ei_hbm.at[:, pl.ds(c * C, C)],
                              ebufs[par], esems[par]).start()

    def drain_edges(c, par):
        pltpu.make_async_copy(ei_hbm.at[:, pl.ds(c * C, C)],
                              ebufs[par], esems[par]).wait()

    def fire_rows(par, boff):
        pltpu.make_async_copy(
            spq.at[msrcs[par].at[pl.ds(boff, G)]],
            rowss[par], gsems[par]).start()

    def drain_rows(par):
        pltpu.make_async_copy(
            spq.at[msrcs[par].at[pl.ds(0, G)]],
            rowss[par], gsems[par]).wait()

    def scan_chunk(par):
        eb, mdl, msrc = ebufs[par], mdls[par], msrcs[par]

        def scan_body(it, cnt):
            for u in range(4):
                base = it * (4 * L) + u * L
                d = eb[1, pl.ds(base, L)]
                s = eb[0, pl.ds(base, L)]
                m = (d >= lo) & (d < hi)
                pc = plsc.all_reduce_population_count(m)
                pcs = pc[0] if pc.ndim else pc
                pos = plsc.cumsum(m.astype(jnp.int32))
                idx = cnt + pos - 1
                plsc.store_scatter(mdl, [idx], (d - lo) * DG, mask=m)
                plsc.store_scatter(msrc, [idx], s, mask=m)
                cnt = cnt + pcs
            return cnt

        cnt = lax.fori_loop(0, C // (4 * L), scan_body, jnp.int32(0))
        mdl[pl.ds(cnt, L)] = jnp.full((L,), DUMMY, jnp.int32)

        @pl.when(cnt > 0)
        def _():
            fire_rows(par, 0)
        return cnt

    def process_batch(par, boff, nedge):
        mdl, rows = mdls[par], rowss[par]
        nblk = (nedge + (L - 1)) // L

        def blk_body(blk, _):
            dv = mdl[pl.ds(boff + blk * L, L)]
            rbase = blk * L
            for i in range(L):
                off = dv[i]
                for j in range(DG // L):
                    sl = pl.ds(off + j * L, L)
                    acc[sl] = jnp.maximum(acc[sl], rows[rbase + i, pl.ds(j * L, L)])
            return 0
        lax.fori_loop(0, nblk, blk_body, 0)

    def process_chunk(par, cnt):
        @pl.when(cnt > 0)
        def _():
            drain_rows(par)
            process_batch(par, 0, jnp.minimum(cnt, G))
            nb = (cnt + (G - 1)) // G

            def more_body(b, _):
                fire_rows(par, b * G)
                drain_rows(par)
                process_batch(par, b * G, jnp.minimum(cnt - b * G, G))
                return 0
            lax.fori_loop(1, nb, more_body, 0)

    fire_edges(0, 0)
    fire_edges(1, 1)
    drain_edges(0, 0)
    cnt0 = scan_chunk(0)

    def half_iter(i, h, cnt_cur):
        @pl.when(i + 2 < NCHUNK)
        def _():
            fire_edges(i + 2, h)

        def do_scan():
            drain_edges(i + 1, 1 - h)
            return scan_chunk(1 - h)
        cnt_next = lax.cond(i + 1 < NCHUNK, do_scan, lambda: jnp.int32(0))

        process_chunk(h, cnt_cur)
        return cnt_next

    def pair_body(k, cnt_cur):
        cnt_cur = half_iter(2 * k, 0, cnt_cur)
        cnt_cur = half_iter(2 * k + 1, 1, cnt_cur)
        return cnt_cur

    lax.fori_loop(0, NCHUNK // 2, pair_body, cnt0)

    pltpu.sync_copy(acc.at[pl.ds(0, NPW * DG)],
                    out_hbm.at[pl.ds(wid * NPW * DG, NPW * DG)])


_segmax = pl.kernel(
    _segmax_body,
    out_type=jax.ShapeDtypeStruct((NPAD * DG,), jnp.float32),
    mesh=plsc.VectorSubcoreMesh(core_axis_name="c", subcore_axis_name="s"),
    scratch_types=(
        [pltpu.VMEM((2, C), jnp.int32)] * 2
        + [pltpu.VMEM((C + L,), jnp.int32)] * 4
        + [pltpu.VMEM((G, DG), jnp.float32)] * 2
        + [pltpu.VMEM(((NPW + 1) * DG,), jnp.float32)]
        + [pltpu.VMEM_SHARED((N, DG), jnp.float32)]
        + [pltpu.SemaphoreType.DMA] * 5
    ),
    compiler_params=pltpu.CompilerParams(
        use_tc_tiling_on_sc=False, needs_layout_passes=False),
)


BR = 1000


def _tca_body(x_ref, we_ref, be_ref, w1_ref, b1_ref, p_ref, q_ref):
    h = jnp.maximum(x_ref[...] @ we_ref[...] + be_ref[...], 0.0)
    w1 = w1_ref[...]
    wt, wb = w1[:D], w1[D:]
    q_ref[...] = h @ wb
    p_ref[...] = h @ (wt - wb) + b1_ref[...]


def _epilogue(p, s):
    v = p + s
    v = jnp.where(v >= 0.0, v, 0.01 * v)
    return jnp.where(jnp.isfinite(v), v, 0.0)


def _tcb_body(p_ref, s_ref, w2_ref, b2_ref, p2_ref, q2_ref):
    h1 = _epilogue(p_ref[...], s_ref[...])
    w2 = w2_ref[...]
    wt, wb = w2[:DG], w2[DG:]
    q2_ref[...] = h1 @ wb
    p2_ref[...] = h1 @ (wt - wb) + b2_ref[...]


def _tcc_body(p_ref, s_ref, wc_ref, bc_ref, out_ref):
    h2 = _epilogue(p_ref[...], s_ref[...])
    out_ref[...] = h2 @ wc_ref[...] + bc_ref[...]


def _full(shape):
    return pl.BlockSpec(shape, lambda i: (0, 0))


def _rows(w):
    return pl.BlockSpec((BR, w), lambda i: (i, 0))


_tca = pl.pallas_call(
    _tca_body,
    grid=(N // BR,),
    in_specs=[_rows(D), _full((D, D)), _full((1, D)),
              _full((2 * D, DG)), _full((1, DG))],
    out_specs=[_rows(DG), _rows(DG)],
    out_shape=[jax.ShapeDtypeStruct((N, DG), jnp.float32)] * 2,
)

_tcb = pl.pallas_call(
    _tcb_body,
    grid=(N // BR,),
    in_specs=[_rows(DG), _rows(DG), _full((2 * DG, DG)), _full((1, DG))],
    out_specs=[_rows(DG), _rows(DG)],
    out_shape=[jax.ShapeDtypeStruct((N, DG), jnp.float32)] * 2,
)

_tcc = pl.pallas_call(
    _tcc_body,
    grid=(N // BR,),
    in_specs=[_rows(DG), _rows(DG), _full((DG, NUM_CLS)),
              _full((1, NUM_CLS))],
    out_specs=_rows(NUM_CLS),
    out_shape=jax.ShapeDtypeStruct((N, NUM_CLS), jnp.float32),
)


@jax.jit
def kernel(x, edge_index, W_embed, b_embed, W1, b1, W2, b2, Wc, bc):
    p1, q1 = _tca(x, W_embed, b_embed.reshape(1, D), W1, b1.reshape(1, DG))
    s1 = _segmax(q1, edge_index).reshape(NPAD, DG)[:N]
    p2, q2 = _tcb(p1, s1, W2, b2.reshape(1, DG))
    s2 = _segmax(q2, edge_index).reshape(NPAD, DG)[:N]
    return _tcc(p2, s2, Wc, bc.reshape(1, NUM_CLS))


# speedup vs baseline: 3.3278x; 1.6726x over previous
import functools

import jax
import jax.numpy as jnp
from jax import lax
from jax.experimental import pallas as pl
from jax.experimental.pallas import tpu as pltpu
import jax.experimental.pallas.tpu_sc as plsc

N = 10000
D = 128
E = 320000
DG = 64
NUM_CLS = 16

NC = 2
NS = 16
NW = NC * NS
L = 16
NPW = (N + NW - 1) // NW
NPAD = NW * NPW
C = 3200
NCHUNK = E // C
G = 128
DUMMY = NPW * DG
NEG_INF = float("-inf")


def _segmax_body(q_hbm, ei_hbm, out_hbm,
                 eb0, eb1, mdl0, msrc0, mdl1, msrc1, rows0, rows1, acc,
                 spq, esem0, esem1, gsem0, gsem1, ssem):
    wid = lax.axis_index("s") * NC + lax.axis_index("c")
    sid = lax.axis_index("s")
    lo = wid * NPW
    hi = lo + NPW

    ebufs = (eb0, eb1)
    mdls = (mdl0, mdl1)
    msrcs = (msrc0, msrc1)
    rowss = (rows0, rows1)
    esems = (esem0, esem1)
    gsems = (gsem0, gsem1)

    QS = N // NS
    pltpu.async_copy(q_hbm.at[pl.ds(sid * QS, QS)],
                     spq.at[pl.ds(sid * QS, QS)], ssem).wait()

    def initacc_body(i, _):
        acc[pl.ds(i * L, L)] = jnp.full((L,), NEG_INF, jnp.float32)
        return 0
    lax.fori_loop(0, (NPW + 1) * DG // L, initacc_body, 0)

    def initsrc_body(i, _):
        z = jnp.zeros((L,), jnp.int32)
        msrc0[pl.ds(i * L, L)] = z
        msrc1[pl.ds(i * L, L)] = z
        return 0
    lax.fori_loop(0, (C + L) // L, initsrc_body, 0)
    plsc.subcore_barrier()

    def fire_edges(c, par):
        pltpu.make_async_copy(ei_hbm.at[:, pl.ds(c * C, C)],
                              ebufs[par], esems[par]).start()

    def drain_edges(c, par):
        pltpu.make_async_copy(ei_hbm.at[:, pl.ds(c * C, C)],
                              ebufs[par], esems[par]).wait()

    def fire_rows(par, boff):
        pltpu.make_async_copy(
            spq.at[msrcs[par].at[pl.ds(boff, G)]],
            rowss[par], gsems[par]).start()

    def drain_rows(par):
        pltpu.make_async_copy(
            spq.at[msrcs[par].at[pl.ds(0, G)]],
            rowss[par], gsems[par]).wait()

    def scan_chunk(par):
        eb, mdl, msrc = ebufs[par], mdls[par], msrcs[par]

        @plsc.parallel_loop(0, C // L, unroll=4, carry=jnp.int32(0))
        def cnt(it, cnt):
            base = it * L
            d = eb[1, pl.ds(base, L)]
            s = eb[0, pl.ds(base, L)]
            m = (d >= lo) & (d < hi)
            pc = plsc.all_reduce_population_count(m)
            pcs = pc[0] if pc.ndim else pc
            pos = plsc.cumsum(m.astype(jnp.int32))
            idx = cnt + pos - 1
            plsc.store_scatter(mdl, [idx], (d - lo) * DG, mask=m)
            plsc.store_scatter(msrc, [idx], s, mask=m)
            return cnt + pcs
        mdl[pl.ds(cnt, L)] = jnp.full((L,), DUMMY, jnp.int32)

        @pl.when(cnt > 0)
        def _():
            fire_rows(par, 0)
        return cnt

    def process_batch(par, boff, nedge):
        mdl, rows = mdls[par], rowss[par]
        nblk = (nedge + (L - 1)) // L

        def blk_body(blk, _):
            dv = mdl[pl.ds(boff + blk * L, L)]
            rbase = blk * L
            for i in range(L):
                off = dv[i]
                for j in range(DG // L):
                    sl = pl.ds(off + j * L, L)
                    acc[sl] = jnp.maximum(acc[sl], rows[rbase + i, pl.ds(j * L, L)])
            return 0
        lax.fori_loop(0, nblk, blk_body, 0)

    def process_chunk(par, cnt):
        @pl.when(cnt > 0)
        def _():
            drain_rows(par)
            process_batch(par, 0, jnp.minimum(cnt, G))
            nb = (cnt + (G - 1)) // G

            def more_body(b, _):
                fire_rows(par, b * G)
                drain_rows(par)
                process_batch(par, b * G, jnp.minimum(cnt - b * G, G))
                return 0
            lax.fori_loop(1, nb, more_body, 0)

    fire_edges(0, 0)
    fire_edges(1, 1)
    drain_edges(0, 0)
    cnt0 = scan_chunk(0)

    def half_iter(i, h, cnt_cur):
        @pl.when(i + 2 < NCHUNK)
        def _():
            fire_edges(i + 2, h)

        def do_scan():
            drain_edges(i + 1, 1 - h)
            return scan_chunk(1 - h)
        cnt_next = lax.cond(i + 1 < NCHUNK, do_scan, lambda: jnp.int32(0))

        process_chunk(h, cnt_cur)
        return cnt_next

    def pair_body(k, cnt_cur):
        cnt_cur = half_iter(2 * k, 0, cnt_cur)
        cnt_cur = half_iter(2 * k + 1, 1, cnt_cur)
        return cnt_cur

    lax.fori_loop(0, NCHUNK // 2, pair_body, cnt0)

    pltpu.sync_copy(acc.at[pl.ds(0, NPW * DG)],
                    out_hbm.at[pl.ds(wid * NPW * DG, NPW * DG)])


_segmax = pl.kernel(
    _segmax_body,
    out_type=jax.ShapeDtypeStruct((NPAD * DG,), jnp.float32),
    mesh=plsc.VectorSubcoreMesh(core_axis_name="c", subcore_axis_name="s"),
    scratch_types=(
        [pltpu.VMEM((2, C), jnp.int32)] * 2
        + [pltpu.VMEM((C + L,), jnp.int32)] * 4
        + [pltpu.VMEM((G, DG), jnp.float32)] * 2
        + [pltpu.VMEM(((NPW + 1) * DG,), jnp.float32)]
        + [pltpu.VMEM_SHARED((N, DG), jnp.float32)]
        + [pltpu.SemaphoreType.DMA] * 5
    ),
    compiler_params=pltpu.CompilerParams(
        use_tc_tiling_on_sc=False, needs_layout_passes=False),
)


BR = 1000


def _tca_body(x_ref, we_ref, be_ref, w1_ref, b1_ref, p_ref, q_ref):
    h = jnp.maximum(x_ref[...] @ we_ref[...] + be_ref[...], 0.0)
    w1 = w1_ref[...]
    wt, wb = w1[:D], w1[D:]
    q_ref[...] = h @ wb
    p_ref[...] = h @ (wt - wb) + b1_ref[...]


def _epilogue(p, s):
    v = p + s
    v = jnp.where(v >= 0.0, v, 0.01 * v)
    return jnp.where(jnp.isfinite(v), v, 0.0)


def _tcb_body(p_ref, s_ref, w2_ref, b2_ref, p2_ref, q2_ref):
    h1 = _epilogue(p_ref[...], s_ref[...])
    w2 = w2_ref[...]
    wt, wb = w2[:DG], w2[DG:]
    q2_ref[...] = h1 @ wb
    p2_ref[...] = h1 @ (wt - wb) + b2_ref[...]


def _tcc_body(p_ref, s_ref, wc_ref, bc_ref, out_ref):
    h2 = _epilogue(p_ref[...], s_ref[...])
    out_ref[...] = h2 @ wc_ref[...] + bc_ref[...]


def _full(shape):
    return pl.BlockSpec(shape, lambda i: (0, 0))


def _rows(w):
    return pl.BlockSpec((BR, w), lambda i: (i, 0))


_tca = pl.pallas_call(
    _tca_body,
    grid=(N // BR,),
    in_specs=[_rows(D), _full((D, D)), _full((1, D)),
              _full((2 * D, DG)), _full((1, DG))],
    out_specs=[_rows(DG), _rows(DG)],
    out_shape=[jax.ShapeDtypeStruct((N, DG), jnp.float32)] * 2,
)

_tcb = pl.pallas_call(
    _tcb_body,
    grid=(N // BR,),
    in_specs=[_rows(DG), _rows(DG), _full((2 * DG, DG)), _full((1, DG))],
    out_specs=[_rows(DG), _rows(DG)],
    out_shape=[jax.ShapeDtypeStruct((N, DG), jnp.float32)] * 2,
)

_tcc = pl.pallas_call(
    _tcc_body,
    grid=(N // BR,),
    in_specs=[_rows(DG), _rows(DG), _full((DG, NUM_CLS)),
              _full((1, NUM_CLS))],
    out_specs=_rows(NUM_CLS),
    out_shape=jax.ShapeDtypeStruct((N, NUM_CLS), jnp.float32),
)


@jax.jit
def kernel(x, edge_index, W_embed, b_embed, W1, b1, W2, b2, Wc, bc):
    p1, q1 = _tca(x, W_embed, b_embed.reshape(1, D), W1, b1.reshape(1, DG))
    s1 = _segmax(q1, edge_index).reshape(NPAD, DG)[:N]
    p2, q2 = _tcb(p1, s1, W2, b2.reshape(1, DG))
    s2 = _segmax(q2, edge_index).reshape(NPAD, DG)[:N]
    return _tcc(p2, s2, Wc, bc.reshape(1, NUM_CLS))
